# Initial kernel scaffold; baseline (speedup 1.0000x reference)
import jax
import jax.numpy as jnp
from jax.experimental import pallas as pl


def kernel(sparse_inputs, dense_inputs, emb_tables, dense_weights):
    raise NotImplementedError("write your pallas kernel here")


# speedup vs baseline: 1.3719x; 1.3719x over previous
import functools

import jax
import jax.numpy as jnp
from jax import lax
from jax.experimental import pallas as pl
from jax.experimental.pallas import tpu as pltpu
from jax.experimental.pallas import tpu_sc as plsc

NW = 32


def _onerec_sc(F, B, V, D, CHUNK):
    mesh = plsc.VectorSubcoreMesh(core_axis_name="c", subcore_axis_name="s")
    NJ = CHUNK // 16

    @functools.partial(
        pl.kernel,
        mesh=mesh,
        out_type=jax.ShapeDtypeStruct((NW, CHUNK), jnp.float32),
        scratch_types=[
            pltpu.VMEM((F * CHUNK,), jnp.int32),
            pltpu.VMEM((F * CHUNK,), jnp.float32),
            pltpu.VMEM((D * CHUNK,), jnp.float32),
            pltpu.VMEM((D * 16,), jnp.float32),
            pltpu.VMEM((CHUNK,), jnp.float32),
            pltpu.SemaphoreType.DMA,
        ],
    )
    def k(table_hbm, idx_hbm, dense_hbm, w_hbm, out_hbm,
          idx_v, vals_v, dense_v, w_v, out_v, sem):
        wid = lax.axis_index("s") * 2 + lax.axis_index("c")
        pltpu.sync_copy(idx_hbm.at[wid], idx_v)
        pltpu.sync_copy(dense_hbm.at[wid], dense_v)
        pltpu.sync_copy(w_hbm, w_v)
        pltpu.async_copy(table_hbm.at[idx_v], vals_v, sem).wait()

        def jbody(j, _):
            acc = jnp.zeros((16,), jnp.float32)

            def fbody(f, a):
                return a + vals_v[pl.ds(f * CHUNK + j * 16, 16)]

            acc = lax.fori_loop(0, F, fbody, acc)

            def dbody(d, a):
                return a + (dense_v[pl.ds(d * CHUNK + j * 16, 16)]
                            * w_v[pl.ds(d * 16, 16)])

            acc = lax.fori_loop(0, D, dbody, acc)
            out_v[pl.ds(j * 16, 16)] = acc
            return 0

        lax.fori_loop(0, NJ, jbody, 0)
        pltpu.sync_copy(out_v, out_hbm.at[wid])

    return k


def kernel(sparse_inputs, dense_inputs, emb_tables, dense_weights):
    F, B, _ = sparse_inputs.shape
    D = dense_inputs.shape[0]
    V = emb_tables.shape[1]
    CHUNK = B // NW

    idx = sparse_inputs.reshape(F, B).astype(jnp.int32)
    idx = idx + (jnp.arange(F, dtype=jnp.int32) * V)[:, None]
    idx = idx.reshape(F, NW, CHUNK).transpose(1, 0, 2).reshape(NW, F * CHUNK)

    dense = dense_inputs.reshape(D, NW, CHUNK).transpose(1, 0, 2)
    dense = dense.reshape(NW, D * CHUNK)

    table = emb_tables.reshape(F * V)
    wrep = jnp.repeat(dense_weights.reshape(D), 16)

    out = _onerec_sc(F, B, V, D, CHUNK)(table, idx, dense, wrep)
    return out.reshape(B, 1)
